# baseline (speedup 1.0000x reference)
import functools

import jax
import jax.numpy as jnp
from jax import lax
from jax.experimental import pallas as pl
from jax.experimental.pallas import tpu as pltpu
from jax.experimental.pallas import tpu_sc as plsc

_NC = 2
_NS = 16
_NW = _NC * _NS

_CH = 128


def _sc_gather(table, idx3):
    nw, n_ch, ch = idx3.shape
    d = table.shape[1]
    o_per_w = n_ch * (ch // 2)
    n_orows = nw * o_per_w

    mesh = plsc.VectorSubcoreMesh(
        core_axis_name="c", subcore_axis_name="s",
        num_cores=_NC, num_subcores=_NS)

    nbuf = 2
    assert n_ch % nbuf == 0

    @functools.partial(
        pl.kernel,
        mesh=mesh,
        out_type=jax.ShapeDtypeStruct((n_orows, d), jnp.int32),
        scratch_types=(
            [pltpu.VMEM((n_ch, ch), jnp.int32)]
            + [pltpu.VMEM((ch, d), jnp.int32)] * nbuf
            + [pltpu.VMEM((ch // 2, d), jnp.int32)] * nbuf
            + [pltpu.SemaphoreType.DMA] * (2 * nbuf)
        ),
    )
    def k(table_hbm, idx_hbm, out_hbm, idx_v, *scratch):
        bufs = scratch[:nbuf]
        obufs = scratch[nbuf:2 * nbuf]
        gsems = scratch[2 * nbuf:3 * nbuf]
        dsems = scratch[3 * nbuf:]
        wid = lax.axis_index("s") * _NC + lax.axis_index("c")
        base = wid * o_per_w
        pltpu.sync_copy(idx_hbm.at[wid], idx_v)

        def startg(j, t):
            pltpu.async_copy(table_hbm.at[idx_v.at[j]], bufs[t], gsems[t])

        def waitg(j, t):
            pltpu.make_async_copy(
                table_hbm.at[idx_v.at[j]], bufs[t], gsems[t]).wait()

        def convert(t):
            rnd = jnp.int32(0x8000)
            himask = jnp.int32(-65536)

            def cvt(q, carry):
                r = 2 * q
                for cc in range(d // 16):
                    c = cc * 16
                    a = bufs[t][r, pl.ds(c, 16)]
                    b = bufs[t][r + 1, pl.ds(c, 16)]
                    ra = lax.shift_right_logical(a + rnd, 16)
                    rb = (b + rnd) & himask
                    obufs[t][q, pl.ds(c, 16)] = ra | rb
                return carry

            lax.fori_loop(0, ch // 2, cvt, 0)

        def startd(j, t):
            pltpu.async_copy(
                obufs[t], out_hbm.at[pl.ds(base + j * (ch // 2), ch // 2)],
                dsems[t])

        def waitd(j, t):
            pltpu.make_async_copy(
                obufs[t], out_hbm.at[pl.ds(base + j * (ch // 2), ch // 2)],
                dsems[t]).wait()

        for t in range(nbuf):
            startg(t, t)

        def body(jj, carry):
            j = nbuf * jj
            for t in range(nbuf):
                waitg(j + t, t)
                @pl.when(jj > 0)
                def _(t=t):
                    waitd(j + t - nbuf, t)
                convert(t)
                startd(j + t, t)
                @pl.when(j + nbuf + t < n_ch)
                def _(t=t):
                    startg(j + nbuf + t, t)
            return carry

        lax.fori_loop(0, n_ch // nbuf, body, 0)
        for t in range(nbuf):
            waitd(n_ch - nbuf + t, t)

    return k(table, idx3)


def _tc_mlp(x3, w1, b1, w2, b2, w3, b3, tb=256, lg=16):
    bpairs, seq, d = x3.shape
    batch = 2 * bpairs
    u1 = w1.shape[1]
    u2 = w2.shape[1]
    nb = batch // tb
    nk = seq // lg

    def body(x_ref, w1_ref, b1_ref, w2_ref, b2_ref, w3_ref, b3_ref,
             out_ref, acc_ref):
        k = pl.program_id(0)
        i = pl.program_id(1)
        xb = jnp.concatenate(
            [pltpu.bitcast(x_ref[:, j, :], jnp.bfloat16) for j in range(lg)],
            axis=1)
        part = jnp.dot(xb, w1_ref[...], preferred_element_type=jnp.float32)
        sl = pl.ds(i * tb, tb)

        @pl.when(k == 0)
        def _():
            acc_ref[sl, :] = part

        @pl.when(k > 0)
        def _():
            acc_ref[sl, :] += part

        @pl.when(k == nk - 1)
        def _():
            h1 = jnp.maximum(acc_ref[sl, :] + b1_ref[...], 0.0)
            h2 = jnp.maximum(
                jnp.dot(h1, w2_ref[...], preferred_element_type=jnp.float32)
                + b2_ref[...], 0.0)
            z = (jnp.dot(h2, w3_ref[...], preferred_element_type=jnp.float32)
                 + b3_ref[...])
            out_ref[...] = jax.nn.sigmoid(z)

    return pl.pallas_call(
        body,
        grid=(nk, nb),
        in_specs=[
            pl.BlockSpec((tb // 2, lg, d), lambda k, i: (i, k, 0)),
            pl.BlockSpec((lg * d, u1), lambda k, i: (k, 0)),
            pl.BlockSpec((1, u1), lambda k, i: (0, 0)),
            pl.BlockSpec((u1, u2), lambda k, i: (0, 0)),
            pl.BlockSpec((1, u2), lambda k, i: (0, 0)),
            pl.BlockSpec((u2, 1), lambda k, i: (0, 0)),
            pl.BlockSpec((1, 1), lambda k, i: (0, 0)),
        ],
        out_specs=pl.BlockSpec((tb, 1), lambda k, i: (i, 0)),
        out_shape=jax.ShapeDtypeStruct((batch, 1), jnp.float32),
        scratch_shapes=[pltpu.VMEM((batch, u1), jnp.float32)],
        compiler_params=pltpu.CompilerParams(
            dimension_semantics=("arbitrary", "arbitrary")),
    )(x3, w1, b1, w2, b2, w3, b3)


_CHUNKS = 4


def kernel(features, emb_table, W1, b1, W2, b2, W3, b3):
    batch, seq = features.shape
    d = emb_table.shape[1]
    bc = batch // _CHUNKS
    idx = (features.astype(jnp.int32)
           .reshape(_CHUNKS, bc // 2, 2, 2, seq // 2)
           .transpose(0, 1, 3, 4, 2)
           .reshape(_CHUNKS, _NW, -1, _CH))
    w1b = W1.astype(jnp.bfloat16)
    b1r = b1.reshape(1, -1)
    b2r = b2.reshape(1, -1)
    b3r = b3.reshape(1, 1)

    table_i32 = lax.bitcast_convert_type(emb_table, jnp.int32)
    rows = [_sc_gather(table_i32, idx[c]) for c in range(_CHUNKS)]
    outs = [
        _tc_mlp(rows[c].reshape(bc // 2, seq, d), w1b, b1r, W2, b2r, W3, b3r)
        for c in range(_CHUNKS)
    ]
    return jnp.concatenate(outs, axis=0)


# speedup vs baseline: 1.0519x; 1.0519x over previous
import functools

import jax
import jax.numpy as jnp
from jax import lax
from jax.experimental import pallas as pl
from jax.experimental.pallas import tpu as pltpu
from jax.experimental.pallas import tpu_sc as plsc

_NC = 2
_NS = 16
_NW = _NC * _NS

_CH = 128


def _sc_gather(table, idx3):
    nw, n_ch, ch = idx3.shape
    d = table.shape[1]
    o_per_w = n_ch * (ch // 2)
    n_orows = nw * o_per_w

    mesh = plsc.VectorSubcoreMesh(
        core_axis_name="c", subcore_axis_name="s",
        num_cores=_NC, num_subcores=_NS)

    nbuf = 2
    assert n_ch % nbuf == 0

    @functools.partial(
        pl.kernel,
        mesh=mesh,
        out_type=jax.ShapeDtypeStruct((n_orows, d), jnp.int32),
        scratch_types=(
            [pltpu.VMEM((n_ch, ch), jnp.int32)]
            + [pltpu.VMEM((ch, d), jnp.int32)] * nbuf
            + [pltpu.VMEM((ch // 2, d), jnp.int32)] * nbuf
            + [pltpu.SemaphoreType.DMA] * (2 * nbuf)
        ),
    )
    def k(table_hbm, idx_hbm, out_hbm, idx_v, *scratch):
        bufs = scratch[:nbuf]
        obufs = scratch[nbuf:2 * nbuf]
        gsems = scratch[2 * nbuf:3 * nbuf]
        dsems = scratch[3 * nbuf:]
        wid = lax.axis_index("s") * _NC + lax.axis_index("c")
        base = wid * o_per_w
        pltpu.sync_copy(idx_hbm.at[wid], idx_v)

        def startg(j, t):
            pltpu.async_copy(table_hbm.at[idx_v.at[j]], bufs[t], gsems[t])

        def waitg(j, t):
            pltpu.make_async_copy(
                table_hbm.at[idx_v.at[j]], bufs[t], gsems[t]).wait()

        def convert(t):
            rnd = jnp.int32(0x8000)
            himask = jnp.int32(-65536)

            @functools.partial(plsc.parallel_loop, 0, ch // 2, unroll=4)
            def _(q):
                r = 2 * q
                for cc in range(d // 16):
                    c = cc * 16
                    a = bufs[t][r, pl.ds(c, 16)]
                    b = bufs[t][r + 1, pl.ds(c, 16)]
                    ra = lax.shift_right_logical(a + rnd, 16)
                    rb = (b + rnd) & himask
                    obufs[t][q, pl.ds(c, 16)] = ra | rb

        def startd(j, t):
            pltpu.async_copy(
                obufs[t], out_hbm.at[pl.ds(base + j * (ch // 2), ch // 2)],
                dsems[t])

        def waitd(j, t):
            pltpu.make_async_copy(
                obufs[t], out_hbm.at[pl.ds(base + j * (ch // 2), ch // 2)],
                dsems[t]).wait()

        for t in range(nbuf):
            startg(t, t)

        def body(jj, carry):
            j = nbuf * jj
            for t in range(nbuf):
                waitg(j + t, t)
                @pl.when(jj > 0)
                def _(t=t):
                    waitd(j + t - nbuf, t)
                convert(t)
                startd(j + t, t)
                @pl.when(j + nbuf + t < n_ch)
                def _(t=t):
                    startg(j + nbuf + t, t)
            return carry

        lax.fori_loop(0, n_ch // nbuf, body, 0)
        for t in range(nbuf):
            waitd(n_ch - nbuf + t, t)

    return k(table, idx3)


def _tc_mlp(x3, w1, b1, w2, b2, w3, b3, tb=256, lg=16):
    bpairs, seq, d = x3.shape
    batch = 2 * bpairs
    u1 = w1.shape[1]
    u2 = w2.shape[1]
    nb = batch // tb
    nk = seq // lg

    def body(x_ref, w1_ref, b1_ref, w2_ref, b2_ref, w3_ref, b3_ref,
             out_ref, acc_ref):
        k = pl.program_id(0)
        i = pl.program_id(1)
        xb = jnp.concatenate(
            [pltpu.bitcast(x_ref[:, j, :], jnp.bfloat16) for j in range(lg)],
            axis=1)
        part = jnp.dot(xb, w1_ref[...], preferred_element_type=jnp.float32)
        sl = pl.ds(i * tb, tb)

        @pl.when(k == 0)
        def _():
            acc_ref[sl, :] = part

        @pl.when(k > 0)
        def _():
            acc_ref[sl, :] += part

        @pl.when(k == nk - 1)
        def _():
            h1 = jnp.maximum(acc_ref[sl, :] + b1_ref[...], 0.0)
            h2 = jnp.maximum(
                jnp.dot(h1, w2_ref[...], preferred_element_type=jnp.float32)
                + b2_ref[...], 0.0)
            z = (jnp.dot(h2, w3_ref[...], preferred_element_type=jnp.float32)
                 + b3_ref[...])
            out_ref[...] = jax.nn.sigmoid(z)

    return pl.pallas_call(
        body,
        grid=(nk, nb),
        in_specs=[
            pl.BlockSpec((tb // 2, lg, d), lambda k, i: (i, k, 0)),
            pl.BlockSpec((lg * d, u1), lambda k, i: (k, 0)),
            pl.BlockSpec((1, u1), lambda k, i: (0, 0)),
            pl.BlockSpec((u1, u2), lambda k, i: (0, 0)),
            pl.BlockSpec((1, u2), lambda k, i: (0, 0)),
            pl.BlockSpec((u2, 1), lambda k, i: (0, 0)),
            pl.BlockSpec((1, 1), lambda k, i: (0, 0)),
        ],
        out_specs=pl.BlockSpec((tb, 1), lambda k, i: (i, 0)),
        out_shape=jax.ShapeDtypeStruct((batch, 1), jnp.float32),
        scratch_shapes=[pltpu.VMEM((batch, u1), jnp.float32)],
        compiler_params=pltpu.CompilerParams(
            dimension_semantics=("arbitrary", "arbitrary")),
    )(x3, w1, b1, w2, b2, w3, b3)


_CHUNKS = 4


def kernel(features, emb_table, W1, b1, W2, b2, W3, b3):
    batch, seq = features.shape
    d = emb_table.shape[1]
    bc = batch // _CHUNKS
    idx = (features.astype(jnp.int32)
           .reshape(_CHUNKS, bc // 2, 2, 2, seq // 2)
           .transpose(0, 1, 3, 4, 2)
           .reshape(_CHUNKS, _NW, -1, _CH))
    w1b = W1.astype(jnp.bfloat16)
    b1r = b1.reshape(1, -1)
    b2r = b2.reshape(1, -1)
    b3r = b3.reshape(1, 1)

    table_i32 = lax.bitcast_convert_type(emb_table, jnp.int32)
    rows = [_sc_gather(table_i32, idx[c]) for c in range(_CHUNKS)]
    outs = [
        _tc_mlp(rows[c].reshape(bc // 2, seq, d), w1b, b1r, W2, b2r, W3, b3r)
        for c in range(_CHUNKS)
    ]
    return jnp.concatenate(outs, axis=0)


# speedup vs baseline: 2.5146x; 2.3905x over previous
import functools

import jax
import jax.numpy as jnp
from jax import lax
from jax.experimental import pallas as pl
from jax.experimental.pallas import tpu as pltpu
from jax.experimental.pallas import tpu_sc as plsc

_NC = 2
_NS = 16
_NW = _NC * _NS

_CH = 128


def _sc_gather(table, idx3):
    nw, n_ch, ch = idx3.shape
    d = table.shape[1]
    b_per_w = n_ch * ch
    n_rows = nw * b_per_w

    mesh = plsc.VectorSubcoreMesh(
        core_axis_name="c", subcore_axis_name="s",
        num_cores=_NC, num_subcores=_NS)

    nbuf = 4
    assert n_ch % nbuf == 0

    @functools.partial(
        pl.kernel,
        mesh=mesh,
        out_type=jax.ShapeDtypeStruct((n_rows, d), jnp.float32),
        scratch_types=(
            [pltpu.VMEM((n_ch, ch), jnp.int32)]
            + [pltpu.VMEM((ch, d), jnp.float32)] * nbuf
            + [pltpu.SemaphoreType.DMA] * (2 * nbuf)
        ),
    )
    def k(table_hbm, idx_hbm, out_hbm, idx_v, *scratch):
        bufs = scratch[:nbuf]
        gsems = scratch[nbuf:2 * nbuf]
        dsems = scratch[2 * nbuf:]
        wid = lax.axis_index("s") * _NC + lax.axis_index("c")
        base = wid * b_per_w
        pltpu.sync_copy(idx_hbm.at[wid], idx_v)

        def startg(j, t):
            pltpu.async_copy(table_hbm.at[idx_v.at[j]], bufs[t], gsems[t])

        def waitg(j, t):
            pltpu.make_async_copy(
                table_hbm.at[idx_v.at[j]], bufs[t], gsems[t]).wait()

        def startd(j, t):
            pltpu.async_copy(
                bufs[t], out_hbm.at[pl.ds(base + j * ch, ch)], dsems[t])

        def waitd(j, t):
            pltpu.make_async_copy(
                bufs[t], out_hbm.at[pl.ds(base + j * ch, ch)], dsems[t]).wait()

        for t in range(nbuf):
            startg(t, t)

        def body(jj, carry):
            j = nbuf * jj
            for t in range(nbuf):
                waitg(j + t, t)
                startd(j + t, t)
            for t in range(nbuf):
                @pl.when(j + nbuf + t < n_ch)
                def _(t=t):
                    waitd(j + t, t)
                    startg(j + nbuf + t, t)
            return carry

        lax.fori_loop(0, n_ch // nbuf, body, 0)
        for t in range(nbuf):
            waitd(n_ch - nbuf + t, t)

    return k(table, idx3)


def _tc_mlp(x3, w1, b1, w2, b2, w3, b3, tb=256, lg=16):
    batch, seq, d = x3.shape
    u1 = w1.shape[1]
    u2 = w2.shape[1]
    nb = batch // tb
    nk = seq // lg

    def body(x_ref, w1_ref, b1_ref, w2_ref, b2_ref, w3_ref, b3_ref,
             out_ref, acc_ref):
        k = pl.program_id(0)
        i = pl.program_id(1)
        xb = jnp.concatenate(
            [x_ref[:, j, :] for j in range(lg)], axis=1).astype(jnp.bfloat16)
        part = jnp.dot(xb, w1_ref[...], preferred_element_type=jnp.float32)
        sl = pl.ds(i * tb, tb)

        @pl.when(k == 0)
        def _():
            acc_ref[sl, :] = part

        @pl.when(k > 0)
        def _():
            acc_ref[sl, :] += part

        @pl.when(k == nk - 1)
        def _():
            h1 = jnp.maximum(acc_ref[sl, :] + b1_ref[...], 0.0)
            h2 = jnp.maximum(
                jnp.dot(h1, w2_ref[...], preferred_element_type=jnp.float32)
                + b2_ref[...], 0.0)
            z = (jnp.dot(h2, w3_ref[...], preferred_element_type=jnp.float32)
                 + b3_ref[...])
            out_ref[...] = jax.nn.sigmoid(z)

    return pl.pallas_call(
        body,
        grid=(nk, nb),
        in_specs=[
            pl.BlockSpec((tb, lg, d), lambda k, i: (i, k, 0)),
            pl.BlockSpec((lg * d, u1), lambda k, i: (k, 0)),
            pl.BlockSpec((1, u1), lambda k, i: (0, 0)),
            pl.BlockSpec((u1, u2), lambda k, i: (0, 0)),
            pl.BlockSpec((1, u2), lambda k, i: (0, 0)),
            pl.BlockSpec((u2, 1), lambda k, i: (0, 0)),
            pl.BlockSpec((1, 1), lambda k, i: (0, 0)),
        ],
        out_specs=pl.BlockSpec((tb, 1), lambda k, i: (i, 0)),
        out_shape=jax.ShapeDtypeStruct((batch, 1), jnp.float32),
        scratch_shapes=[pltpu.VMEM((batch, u1), jnp.float32)],
        compiler_params=pltpu.CompilerParams(
            dimension_semantics=("arbitrary", "arbitrary")),
    )(x3, w1, b1, w2, b2, w3, b3)


_CHUNKS = 4


def kernel(features, emb_table, W1, b1, W2, b2, W3, b3):
    batch, seq = features.shape
    d = emb_table.shape[1]
    bc = batch // _CHUNKS
    idx = features.astype(jnp.int32).reshape(_CHUNKS, _NW, -1, _CH)
    w1b = W1.astype(jnp.bfloat16)
    b1r = b1.reshape(1, -1)
    b2r = b2.reshape(1, -1)
    b3r = b3.reshape(1, 1)

    rows = [_sc_gather(emb_table, idx[c]) for c in range(_CHUNKS)]
    outs = [
        _tc_mlp(rows[c].reshape(bc, seq, d), w1b, b1r, W2, b2r, W3, b3r)
        for c in range(_CHUNKS)
    ]
    return jnp.concatenate(outs, axis=0)
